# Initial kernel scaffold; baseline (speedup 1.0000x reference)
import jax
import jax.numpy as jnp
from jax.experimental import pallas as pl


def kernel(x, W1, g1, b1, W2, g2, b2, W3, g3, b3, W4, g4, b4, W5, g5, b5, L1, g6, b6, L2, L2b, g7, b7, L3, L3b):
    raise NotImplementedError("write your pallas kernel here")


# speedup vs baseline: 5.9690x; 5.9690x over previous
import functools

import jax
import jax.numpy as jnp
from jax import lax
from jax.experimental import pallas as pl
from jax.experimental.pallas import tpu as pltpu
from jax.experimental.pallas import tpu_sc as plsc

B = 32
N = 1024
K = 20
CP = 128
BN_ROWS = B * N
EPS = 1e-5


_KNN_TR = 256


def _knn_body(f_tile_ref, f_all_ref, xxl_ref, xxs_ref, idx_ref):
    b = pl.program_id(0)
    ft = f_tile_ref[0]
    fa = f_all_ref[0]
    g = lax.dot_general(ft, fa, (((1,), (1,)), ((), ())),
                        preferred_element_type=jnp.float32)
    inner = -2.0 * g
    pd = (-xxl_ref[0]) - inner - xxs_ref[0]
    colid = lax.broadcasted_iota(jnp.int32, (_KNN_TR, N), 1)
    lane32 = lax.broadcasted_iota(jnp.int32, (_KNN_TR, 32), 1)
    buf = jnp.zeros((_KNN_TR, 32), jnp.int32)
    neg_inf = jnp.float32(-jnp.inf)
    for k in range(K):
        m = jnp.max(pd, axis=1, keepdims=True)
        cand = jnp.where(pd == m, colid, N)
        j = jnp.min(cand, axis=1, keepdims=True)
        buf = jnp.where(lane32 == k, j + b * N, buf)
        pd = jnp.where(colid == j, neg_inf, pd)
    idx_ref[0] = buf[:, :K]


def _knn(frows, xx):
    grid = (B, N // _KNN_TR)
    xxt = jnp.transpose(xx, (0, 2, 1))
    return pl.pallas_call(
        _knn_body,
        grid=grid,
        in_specs=[
            pl.BlockSpec((1, _KNN_TR, CP), lambda b, r: (b, r, 0)),
            pl.BlockSpec((1, N, CP), lambda b, r: (b, 0, 0)),
            pl.BlockSpec((1, 1, N), lambda b, r: (b, 0, 0)),
            pl.BlockSpec((1, _KNN_TR, 1), lambda b, r: (b, r, 0)),
        ],
        out_specs=pl.BlockSpec((1, _KNN_TR, K), lambda b, r: (b, r, 0)),
        out_shape=jax.ShapeDtypeStruct((B, N, K), jnp.int32),
    )(frows, frows, xx, xxt)


def _conv_body(z_ref, w_ref, y_ref, s_ref):
    i = pl.program_id(0)
    y = lax.dot_general(z_ref[...], w_ref[...], (((1,), (1,)), ((), ())),
                        preferred_element_type=jnp.float32)
    y_ref[...] = y
    ps = jnp.sum(y, axis=0, keepdims=True)
    pss = jnp.sum(y * y, axis=0, keepdims=True)
    row = lax.broadcasted_iota(jnp.int32, (8, y.shape[1]), 0)
    part = jnp.where(row == 0, ps, jnp.where(row == 1, pss, 0.0))

    @pl.when(i == 0)
    def _():
        s_ref[...] = jnp.zeros_like(s_ref)

    s_ref[...] += part


def _conv(z, w, tm=2048):
    cin, co = w.shape[1], w.shape[0]
    grid = (BN_ROWS // tm,)
    return pl.pallas_call(
        _conv_body,
        grid=grid,
        in_specs=[
            pl.BlockSpec((tm, cin), lambda i: (i, 0)),
            pl.BlockSpec((co, cin), lambda i: (0, 0)),
        ],
        out_specs=[
            pl.BlockSpec((tm, co), lambda i: (i, 0)),
            pl.BlockSpec((8, co), lambda i: (0, 0)),
        ],
        out_shape=[
            jax.ShapeDtypeStruct((BN_ROWS, co), jnp.float32),
            jax.ShapeDtypeStruct((8, co), jnp.float32),
        ],
    )(z, w)


def _conv5_body(z1_ref, z2_ref, z3_ref, z4_ref, w1_ref, w2_ref, w3_ref,
                w4_ref, y_ref, s_ref):
    i = pl.program_id(0)
    dn = (((1,), (1,)), ((), ()))
    y = lax.dot_general(z1_ref[...], w1_ref[...], dn,
                        preferred_element_type=jnp.float32)
    y += lax.dot_general(z2_ref[...], w2_ref[...], dn,
                         preferred_element_type=jnp.float32)
    y += lax.dot_general(z3_ref[...], w3_ref[...], dn,
                         preferred_element_type=jnp.float32)
    y += lax.dot_general(z4_ref[...], w4_ref[...], dn,
                         preferred_element_type=jnp.float32)
    y_ref[...] = y
    ps = jnp.sum(y, axis=0, keepdims=True)
    pss = jnp.sum(y * y, axis=0, keepdims=True)
    row = lax.broadcasted_iota(jnp.int32, (8, y.shape[1]), 0)
    part = jnp.where(row == 0, ps, jnp.where(row == 1, pss, 0.0))

    @pl.when(i == 0)
    def _():
        s_ref[...] = jnp.zeros_like(s_ref)

    s_ref[...] += part


def _conv5(z1, z2, z3, z4, w5, tm=512):
    co = w5.shape[0]
    w5a, w5b, w5c, w5d = w5[:, :128], w5[:, 128:256], w5[:, 256:384], w5[:, 384:]
    grid = (BN_ROWS // tm,)
    return pl.pallas_call(
        _conv5_body,
        grid=grid,
        in_specs=[
            pl.BlockSpec((tm, 128), lambda i: (i, 0)),
            pl.BlockSpec((tm, 128), lambda i: (i, 0)),
            pl.BlockSpec((tm, 128), lambda i: (i, 0)),
            pl.BlockSpec((tm, 256), lambda i: (i, 0)),
            pl.BlockSpec((co, 128), lambda i: (0, 0)),
            pl.BlockSpec((co, 128), lambda i: (0, 0)),
            pl.BlockSpec((co, 128), lambda i: (0, 0)),
            pl.BlockSpec((co, 256), lambda i: (0, 0)),
        ],
        out_specs=[
            pl.BlockSpec((tm, co), lambda i: (i, 0)),
            pl.BlockSpec((8, co), lambda i: (0, 0)),
        ],
        out_shape=[
            jax.ShapeDtypeStruct((BN_ROWS, co), jnp.float32),
            jax.ShapeDtypeStruct((8, co), jnp.float32),
        ],
    )(z1, z2, z3, z4, w5a, w5b, w5c, w5d)


_SC_R = 32


@functools.lru_cache(maxsize=None)
def _make_gather_max(co):
    yw = 128
    mesh = plsc.VectorSubcoreMesh(core_axis_name="c", subcore_axis_name="s")
    rows_per_w = BN_ROWS // 32
    n_chunks = rows_per_w // _SC_R
    idx_rows = _SC_R * K // 128

    idx_rows_w = rows_per_w * K // 128

    @functools.partial(
        pl.kernel,
        mesh=mesh,
        out_type=jax.ShapeDtypeStruct((BN_ROWS, 2 * co), jnp.float32),
        scratch_types=[
            pltpu.VMEM((idx_rows_w, 128), jnp.int32),
            pltpu.VMEM((_SC_R * K, yw), jnp.float32),
            pltpu.VMEM((_SC_R, yw), jnp.float32),
            pltpu.VMEM((_SC_R, 2 * co), jnp.float32),
            pltpu.VMEM((co,), jnp.float32),
            pltpu.VMEM((co,), jnp.float32),
            pltpu.SemaphoreType.DMA,
        ],
    )
    def gm(y_hbm, sc_hbm, sh_hbm, idx_hbm, out_hbm,
           idx_v, gat_v, self_v, z_v, sc_v, sh_v, sem):
        wid = lax.axis_index("s") * 2 + lax.axis_index("c")
        pltpu.sync_copy(sc_hbm, sc_v)
        pltpu.sync_copy(sh_hbm, sh_v)
        pltpu.sync_copy(idx_hbm.at[pl.ds(wid * idx_rows_w, idx_rows_w)], idx_v)

        def chunk(ci, carry):
            row0 = wid * rows_per_w + ci * _SC_R
            cps = []
            for i in range(idx_rows):
                cps.append(pltpu.async_copy(
                    y_hbm.at[idx_v.at[ci * idx_rows + i]],
                    gat_v.at[pl.ds(i * 128, 128)], sem))
            pltpu.sync_copy(y_hbm.at[pl.ds(row0, _SC_R)], self_v)
            for cp in cps:
                cp.wait()

            def row(r, c2):
                for l in range(co // 16):
                    sl = pl.ds(l * 16, 16)
                    acc = gat_v[r * K, sl]
                    for j in range(1, K):
                        acc = jnp.maximum(acc, gat_v[r * K + j, sl])
                    a = sc_v[sl]
                    t = sh_v[sl]
                    fm = acc * a + t
                    fm = jnp.where(fm >= 0.0, fm, 0.2 * fm)
                    fs = self_v[r, sl] * a + t
                    fs = jnp.where(fs >= 0.0, fs, 0.2 * fs)
                    z_v[r, sl] = fs
                    z_v[r, pl.ds(co + l * 16, 16)] = fm - fs
                return c2

            lax.fori_loop(0, _SC_R, row, 0)
            pltpu.sync_copy(z_v, out_hbm.at[pl.ds(row0, _SC_R)])
            return carry

        lax.fori_loop(0, n_chunks, chunk, 0)

    return gm


def _colmax_body(y_ref, sc_ref, sh_ref, o_ref):
    a = y_ref[...] * sc_ref[...] + sh_ref[...]
    m = jnp.max(a, axis=1, keepdims=True)
    m = jnp.where(m >= 0.0, m, 0.2 * m)
    o_ref[...] = jnp.broadcast_to(m, (m.shape[0], 8))


def _colmax(y5, sc5, sh5, tm=1024):
    grid = (BN_ROWS // tm,)
    return pl.pallas_call(
        _colmax_body,
        grid=grid,
        in_specs=[
            pl.BlockSpec((tm, 1024), lambda i: (i, 0)),
            pl.BlockSpec((1, 1024), lambda i: (0, 0)),
            pl.BlockSpec((1, 1024), lambda i: (0, 0)),
        ],
        out_specs=pl.BlockSpec((tm, 8), lambda i: (i, 0)),
        out_shape=jax.ShapeDtypeStruct((BN_ROWS, 8), jnp.float32),
    )(y5, sc5, sh5)


def _fc_body(h_ref, l1_ref, g6_ref, b6_ref, l2_ref, l2b_ref, g7_ref, b7_ref,
             l3_ref, l3b_ref, o_ref):
    dn = (((1,), (1,)), ((), ()))

    def bn_lrelu(y, g, bb):
        mean = jnp.mean(y, axis=0, keepdims=True)
        var = jnp.mean((y - mean) * (y - mean), axis=0, keepdims=True)
        a = (y - mean) / jnp.sqrt(var + EPS) * g + bb
        return jnp.where(a >= 0.0, a, 0.2 * a)

    h1 = bn_lrelu(lax.dot_general(h_ref[...], l1_ref[...], dn,
                                  preferred_element_type=jnp.float32),
                  g6_ref[...], b6_ref[...])
    h2 = bn_lrelu(lax.dot_general(h1, l2_ref[...], dn,
                                  preferred_element_type=jnp.float32)
                  + l2b_ref[...],
                  g7_ref[...], b7_ref[...])
    o_ref[...] = lax.dot_general(h2, l3_ref[...], dn,
                                 preferred_element_type=jnp.float32) + l3b_ref[...]


def _fc(h, l1, g6, b6, l2, l2b, g7, b7, l3, l3b):
    return pl.pallas_call(
        _fc_body,
        out_shape=jax.ShapeDtypeStruct((B, 40), jnp.float32),
    )(h, l1, g6.reshape(1, -1), b6.reshape(1, -1), l2, l2b.reshape(1, -1),
      g7.reshape(1, -1), b7.reshape(1, -1), l3, l3b.reshape(1, -1))


def _affine(stats, g, bta):
    mean = stats[0] / BN_ROWS
    var = stats[1] / BN_ROWS - mean * mean
    sc = g / jnp.sqrt(var + EPS)
    return sc, bta - mean * sc


def kernel(x, W1, g1, b1, W2, g2, b2, W3, g3, b3, W4, g4, b4, W5, g5, b5,
           L1, g6, b6, L2, L2b, g7, b7, L3, L3b):
    xr = jnp.transpose(x, (0, 2, 1))
    xp = jnp.pad(xr, ((0, 0), (0, 0), (0, CP - 3)))
    rows0 = xp.reshape(BN_ROWS, CP)
    w1p = jnp.pad(W1, ((0, 0), (0, CP - 3)))

    def layer(frows, zrows, w, g, bta, gm):
        co = w.shape[0]
        xx = jnp.sum(frows * frows, axis=2).reshape(B, 1, N)
        idx = _knn(frows, xx).reshape(-1, 128)
        wp = w if co == 128 else jnp.pad(w, ((0, 128 - co), (0, 0)))
        y, s = _conv(zrows, wp)
        sc, sh = _affine(s[:, :co], g, bta)
        return gm(y, sc, sh, idx)

    gm64, gm128 = _make_gather_max(64), _make_gather_max(128)
    z1 = layer(xp, rows0, w1p, g1, b1, gm64)
    z2 = layer(z1.reshape(B, N, CP), z1, W2, g2, b2, gm64)
    z3 = layer(z2.reshape(B, N, CP), z2, W3, g3, b3, gm64)
    z4 = layer(z3.reshape(B, N, CP), z3, W4, g4, b4, gm128)

    y5, s5 = _conv5(z1, z2, z3, z4, W5)
    sc5, sh5 = _affine(s5, g5, b5)
    hm = _colmax(y5, sc5.reshape(1, -1), sh5.reshape(1, -1))
    h = hm[:, 0].reshape(B, N)

    return _fc(h, L1, g6, b6, L2, L2b, g7, b7, L3, L3b)


# speedup vs baseline: 7.7876x; 1.3047x over previous
import functools

import jax
import jax.numpy as jnp
from jax import lax
from jax.experimental import pallas as pl
from jax.experimental.pallas import tpu as pltpu
from jax.experimental.pallas import tpu_sc as plsc

B = 32
N = 1024
K = 20
CP = 128
BN_ROWS = B * N
EPS = 1e-5


_KNN_TR = 256


def _knn_body(f_tile_ref, f_all_ref, xxl_ref, xxs_ref, idx_ref):
    b = pl.program_id(0)
    ft = f_tile_ref[0]
    fa = f_all_ref[0]
    g = lax.dot_general(ft, fa, (((1,), (1,)), ((), ())),
                        preferred_element_type=jnp.float32)
    inner = -2.0 * g
    pd = (-xxl_ref[0]) - inner - xxs_ref[0]
    colf = lax.broadcasted_iota(jnp.int32, (_KNN_TR, N), 1).astype(jnp.float32)
    lane32 = lax.broadcasted_iota(jnp.int32, (_KNN_TR, 32), 1)
    buf = jnp.zeros((_KNN_TR, 32), jnp.float32)
    neg_inf = jnp.float32(-jnp.inf)
    big = jnp.float32(2.0 * N)
    for k in range(K):
        m = jnp.max(pd, axis=1, keepdims=True)
        cand = jnp.where(pd == m, colf, big)
        j = jnp.min(cand, axis=1, keepdims=True)
        buf = jnp.where(lane32 == k, j, buf)
        pd = jnp.where(colf == j, neg_inf, pd)
    idx_ref[0] = buf[:, :K].astype(jnp.int32) + b * N


def _knn(frows, xx):
    grid = (B, N // _KNN_TR)
    xxt = jnp.transpose(xx, (0, 2, 1))
    return pl.pallas_call(
        _knn_body,
        grid=grid,
        in_specs=[
            pl.BlockSpec((1, _KNN_TR, CP), lambda b, r: (b, r, 0)),
            pl.BlockSpec((1, N, CP), lambda b, r: (b, 0, 0)),
            pl.BlockSpec((1, 1, N), lambda b, r: (b, 0, 0)),
            pl.BlockSpec((1, _KNN_TR, 1), lambda b, r: (b, r, 0)),
        ],
        out_specs=pl.BlockSpec((1, _KNN_TR, K), lambda b, r: (b, r, 0)),
        out_shape=jax.ShapeDtypeStruct((B, N, K), jnp.int32),
    )(frows, frows, xx, xxt)


def _conv_body(z_ref, w_ref, y_ref, s_ref):
    i = pl.program_id(0)
    y = lax.dot_general(z_ref[...], w_ref[...], (((1,), (1,)), ((), ())),
                        preferred_element_type=jnp.float32)
    y_ref[...] = y
    ps = jnp.sum(y, axis=0, keepdims=True)
    pss = jnp.sum(y * y, axis=0, keepdims=True)
    row = lax.broadcasted_iota(jnp.int32, (8, y.shape[1]), 0)
    part = jnp.where(row == 0, ps, jnp.where(row == 1, pss, 0.0))

    @pl.when(i == 0)
    def _():
        s_ref[...] = jnp.zeros_like(s_ref)

    s_ref[...] += part


def _conv(z, w, tm=2048):
    cin, co = w.shape[1], w.shape[0]
    grid = (BN_ROWS // tm,)
    return pl.pallas_call(
        _conv_body,
        grid=grid,
        in_specs=[
            pl.BlockSpec((tm, cin), lambda i: (i, 0)),
            pl.BlockSpec((co, cin), lambda i: (0, 0)),
        ],
        out_specs=[
            pl.BlockSpec((tm, co), lambda i: (i, 0)),
            pl.BlockSpec((8, co), lambda i: (0, 0)),
        ],
        out_shape=[
            jax.ShapeDtypeStruct((BN_ROWS, co), jnp.float32),
            jax.ShapeDtypeStruct((8, co), jnp.float32),
        ],
    )(z, w)


def _conv5_body(z1_ref, z2_ref, z3_ref, z4_ref, w1_ref, w2_ref, w3_ref,
                w4_ref, y_ref, s_ref):
    i = pl.program_id(0)
    dn = (((1,), (1,)), ((), ()))
    y = lax.dot_general(z1_ref[...], w1_ref[...], dn,
                        preferred_element_type=jnp.float32)
    y += lax.dot_general(z2_ref[...], w2_ref[...], dn,
                         preferred_element_type=jnp.float32)
    y += lax.dot_general(z3_ref[...], w3_ref[...], dn,
                         preferred_element_type=jnp.float32)
    y += lax.dot_general(z4_ref[...], w4_ref[...], dn,
                         preferred_element_type=jnp.float32)
    y_ref[...] = y
    ps = jnp.sum(y, axis=0, keepdims=True)
    pss = jnp.sum(y * y, axis=0, keepdims=True)
    row = lax.broadcasted_iota(jnp.int32, (8, y.shape[1]), 0)
    part = jnp.where(row == 0, ps, jnp.where(row == 1, pss, 0.0))

    @pl.when(i == 0)
    def _():
        s_ref[...] = jnp.zeros_like(s_ref)

    s_ref[...] += part


def _conv5(z1, z2, z3, z4, w5, tm=512):
    co = w5.shape[0]
    w5a, w5b, w5c, w5d = w5[:, :128], w5[:, 128:256], w5[:, 256:384], w5[:, 384:]
    grid = (BN_ROWS // tm,)
    return pl.pallas_call(
        _conv5_body,
        grid=grid,
        in_specs=[
            pl.BlockSpec((tm, 128), lambda i: (i, 0)),
            pl.BlockSpec((tm, 128), lambda i: (i, 0)),
            pl.BlockSpec((tm, 128), lambda i: (i, 0)),
            pl.BlockSpec((tm, 256), lambda i: (i, 0)),
            pl.BlockSpec((co, 128), lambda i: (0, 0)),
            pl.BlockSpec((co, 128), lambda i: (0, 0)),
            pl.BlockSpec((co, 128), lambda i: (0, 0)),
            pl.BlockSpec((co, 256), lambda i: (0, 0)),
        ],
        out_specs=[
            pl.BlockSpec((tm, co), lambda i: (i, 0)),
            pl.BlockSpec((8, co), lambda i: (0, 0)),
        ],
        out_shape=[
            jax.ShapeDtypeStruct((BN_ROWS, co), jnp.float32),
            jax.ShapeDtypeStruct((8, co), jnp.float32),
        ],
    )(z1, z2, z3, z4, w5a, w5b, w5c, w5d)


def _var_body(y_ref, m_ref, s_ref):
    i = pl.program_id(0)
    dy = y_ref[...] - m_ref[...]
    pss = jnp.sum(dy * dy, axis=0, keepdims=True)
    row = lax.broadcasted_iota(jnp.int32, (8, dy.shape[1]), 0)
    part = jnp.where(row == 0, pss, 0.0)

    @pl.when(i == 0)
    def _():
        s_ref[...] = jnp.zeros_like(s_ref)

    s_ref[...] += part


def _varsum(y, mean, tm=2048):
    co = y.shape[1]
    grid = (BN_ROWS // tm,)
    return pl.pallas_call(
        _var_body,
        grid=grid,
        in_specs=[
            pl.BlockSpec((tm, co), lambda i: (i, 0)),
            pl.BlockSpec((1, co), lambda i: (0, 0)),
        ],
        out_specs=pl.BlockSpec((8, co), lambda i: (0, 0)),
        out_shape=jax.ShapeDtypeStruct((8, co), jnp.float32),
    )(y, mean.reshape(1, co))


def _bn_stats(y, s):
    mean = s[0] / BN_ROWS
    vs = _varsum(y, mean)
    var = vs[0] / BN_ROWS
    den = jnp.sqrt(var + EPS)
    return mean, den


_SC_R = 32


@functools.lru_cache(maxsize=None)
def _make_gather_max(co):
    yw = 128
    mesh = plsc.VectorSubcoreMesh(core_axis_name="c", subcore_axis_name="s")
    rows_per_w = BN_ROWS // 32
    n_chunks = rows_per_w // _SC_R
    idx_rows = _SC_R * K // 128

    idx_rows_w = rows_per_w * K // 128

    @functools.partial(
        pl.kernel,
        mesh=mesh,
        out_type=jax.ShapeDtypeStruct((BN_ROWS, 2 * co), jnp.float32),
        scratch_types=[
            pltpu.VMEM((idx_rows_w, 128), jnp.int32),
            pltpu.VMEM((_SC_R * K, yw), jnp.float32),
            pltpu.VMEM((_SC_R, yw), jnp.float32),
            pltpu.VMEM((_SC_R, 2 * co), jnp.float32),
            pltpu.VMEM((co,), jnp.float32),
            pltpu.VMEM((co,), jnp.float32),
            pltpu.VMEM((co,), jnp.float32),
            pltpu.VMEM((co,), jnp.float32),
            pltpu.SemaphoreType.DMA,
        ],
    )
    def gm(y_hbm, mn_hbm, dn_hbm, g_hbm, bb_hbm, idx_hbm, out_hbm,
           idx_v, gat_v, self_v, z_v, mn_v, dn_v, g_v, bb_v, sem):
        wid = lax.axis_index("s") * 2 + lax.axis_index("c")
        pltpu.sync_copy(mn_hbm, mn_v)
        pltpu.sync_copy(dn_hbm, dn_v)
        pltpu.sync_copy(g_hbm, g_v)
        pltpu.sync_copy(bb_hbm, bb_v)
        pltpu.sync_copy(idx_hbm.at[pl.ds(wid * idx_rows_w, idx_rows_w)], idx_v)

        def chunk(ci, carry):
            row0 = wid * rows_per_w + ci * _SC_R
            cps = []
            for i in range(idx_rows):
                cps.append(pltpu.async_copy(
                    y_hbm.at[idx_v.at[ci * idx_rows + i]],
                    gat_v.at[pl.ds(i * 128, 128)], sem))
            pltpu.sync_copy(y_hbm.at[pl.ds(row0, _SC_R)], self_v)
            for cp in cps:
                cp.wait()

            def row(r, c2):
                for l in range(co // 16):
                    sl = pl.ds(l * 16, 16)
                    acc = gat_v[r * K, sl]
                    for j in range(1, K):
                        acc = jnp.maximum(acc, gat_v[r * K + j, sl])
                    mn = mn_v[sl]
                    dn = dn_v[sl]
                    gg = g_v[sl]
                    bb = bb_v[sl]
                    fm = ((acc - mn) / dn) * gg + bb
                    fm = jnp.where(fm >= 0.0, fm, 0.2 * fm)
                    fs = ((self_v[r, sl] - mn) / dn) * gg + bb
                    fs = jnp.where(fs >= 0.0, fs, 0.2 * fs)
                    z_v[r, sl] = fs
                    z_v[r, pl.ds(co + l * 16, 16)] = fm - fs
                return c2

            lax.fori_loop(0, _SC_R, row, 0)
            pltpu.sync_copy(z_v, out_hbm.at[pl.ds(row0, _SC_R)])
            return carry

        lax.fori_loop(0, n_chunks, chunk, 0)

    return gm


def _colmax_body(y_ref, mn_ref, dn_ref, g_ref, b_ref, o_ref):
    a = ((y_ref[...] - mn_ref[...]) / dn_ref[...]) * g_ref[...] + b_ref[...]
    m = jnp.max(a, axis=1, keepdims=True)
    m = jnp.where(m >= 0.0, m, 0.2 * m)
    o_ref[...] = jnp.broadcast_to(m, (m.shape[0], 8))


def _colmax(y5, mn, dn, g, bb, tm=1024):
    grid = (BN_ROWS // tm,)
    vec = pl.BlockSpec((1, 1024), lambda i: (0, 0))
    return pl.pallas_call(
        _colmax_body,
        grid=grid,
        in_specs=[pl.BlockSpec((tm, 1024), lambda i: (i, 0)), vec, vec, vec, vec],
        out_specs=pl.BlockSpec((tm, 8), lambda i: (i, 0)),
        out_shape=jax.ShapeDtypeStruct((BN_ROWS, 8), jnp.float32),
    )(y5, mn.reshape(1, -1), dn.reshape(1, -1), g.reshape(1, -1), bb.reshape(1, -1))


def _fc_body(h_ref, l1_ref, g6_ref, b6_ref, l2_ref, l2b_ref, g7_ref, b7_ref,
             l3_ref, l3b_ref, o_ref):
    dn = (((1,), (1,)), ((), ()))

    def bn_lrelu(y, g, bb):
        mean = jnp.mean(y, axis=0, keepdims=True)
        var = jnp.mean((y - mean) * (y - mean), axis=0, keepdims=True)
        a = (y - mean) / jnp.sqrt(var + EPS) * g + bb
        return jnp.where(a >= 0.0, a, 0.2 * a)

    h1 = bn_lrelu(lax.dot_general(h_ref[...], l1_ref[...], dn,
                                  preferred_element_type=jnp.float32),
                  g6_ref[...], b6_ref[...])
    h2 = bn_lrelu(lax.dot_general(h1, l2_ref[...], dn,
                                  preferred_element_type=jnp.float32)
                  + l2b_ref[...],
                  g7_ref[...], b7_ref[...])
    o_ref[...] = lax.dot_general(h2, l3_ref[...], dn,
                                 preferred_element_type=jnp.float32) + l3b_ref[...]


def _fc(h, l1, g6, b6, l2, l2b, g7, b7, l3, l3b):
    return pl.pallas_call(
        _fc_body,
        out_shape=jax.ShapeDtypeStruct((B, 40), jnp.float32),
    )(h, l1, g6.reshape(1, -1), b6.reshape(1, -1), l2, l2b.reshape(1, -1),
      g7.reshape(1, -1), b7.reshape(1, -1), l3, l3b.reshape(1, -1))


def kernel(x, W1, g1, b1, W2, g2, b2, W3, g3, b3, W4, g4, b4, W5, g5, b5,
           L1, g6, b6, L2, L2b, g7, b7, L3, L3b):
    xr = jnp.transpose(x, (0, 2, 1))
    xp = jnp.pad(xr, ((0, 0), (0, 0), (0, CP - 3)))
    rows0 = xp.reshape(BN_ROWS, CP)
    w1p = jnp.pad(W1, ((0, 0), (0, CP - 3)))

    def layer(frows, zrows, w, g, bta, gm, xx):
        co = w.shape[0]
        idx = _knn(frows, xx).reshape(-1, 128)
        wp = w if co == 128 else jnp.pad(w, ((0, 128 - co), (0, 0)))
        y, s = _conv(zrows, wp)
        mean, den = _bn_stats(y, s)
        return gm(y, mean[:co], den[:co], g, bta, idx)

    def xx_of(z):
        return jnp.sum(z * z, axis=1).reshape(B, 1, N)

    gm64, gm128 = _make_gather_max(64), _make_gather_max(128)
    xx1 = jnp.sum(x ** 2, axis=1, keepdims=True)
    z1 = layer(xp, rows0, w1p, g1, b1, gm64, xx1)
    z2 = layer(z1.reshape(B, N, CP), z1, W2, g2, b2, gm64, xx_of(z1))
    z3 = layer(z2.reshape(B, N, CP), z2, W3, g3, b3, gm64, xx_of(z2))
    z4 = layer(z3.reshape(B, N, CP), z3, W4, g4, b4, gm128, xx_of(z3))

    y5, s5 = _conv5(z1, z2, z3, z4, W5)
    mean5, den5 = _bn_stats(y5, s5)
    hm = _colmax(y5, mean5, den5, g5, b5)
    h = hm[:, 0].reshape(B, N)

    return _fc(h, L1, g6, b6, L2, L2b, g7, b7, L3, L3b)


# speedup vs baseline: 7.8834x; 1.0123x over previous
import functools

import jax
import jax.numpy as jnp
from jax import lax
from jax.experimental import pallas as pl
from jax.experimental.pallas import tpu as pltpu
from jax.experimental.pallas import tpu_sc as plsc

B = 32
N = 1024
K = 20
CP = 128
BN_ROWS = B * N
EPS = 1e-5


_KNN_TR = 512


def _knn_body(f_tile_ref, f_all_ref, xxl_ref, xxs_ref, idx_ref):
    b = pl.program_id(0)
    ft = f_tile_ref[0]
    fa = f_all_ref[0]
    g = lax.dot_general(ft, fa, (((1,), (1,)), ((), ())),
                        preferred_element_type=jnp.float32)
    inner = -2.0 * g
    pd = (-xxl_ref[0]) - inner - xxs_ref[0]
    colf = lax.broadcasted_iota(jnp.int32, (_KNN_TR, N), 1).astype(jnp.float32)
    lane32 = lax.broadcasted_iota(jnp.int32, (_KNN_TR, 32), 1)
    buf = jnp.zeros((_KNN_TR, 32), jnp.float32)
    neg_inf = jnp.float32(-jnp.inf)
    big = jnp.float32(2.0 * N)
    for k in range(K):
        m = jnp.max(pd, axis=1, keepdims=True)
        cand = jnp.where(pd == m, colf, big)
        j = jnp.min(cand, axis=1, keepdims=True)
        buf = jnp.where(lane32 == k, j, buf)
        pd = jnp.where(colf == j, neg_inf, pd)
    idx_ref[0] = buf[:, :K].astype(jnp.int32) + b * N


def _knn(frows, xx):
    grid = (B, N // _KNN_TR)
    xxt = jnp.transpose(xx, (0, 2, 1))
    return pl.pallas_call(
        _knn_body,
        grid=grid,
        in_specs=[
            pl.BlockSpec((1, _KNN_TR, CP), lambda b, r: (b, r, 0)),
            pl.BlockSpec((1, N, CP), lambda b, r: (b, 0, 0)),
            pl.BlockSpec((1, 1, N), lambda b, r: (b, 0, 0)),
            pl.BlockSpec((1, _KNN_TR, 1), lambda b, r: (b, r, 0)),
        ],
        out_specs=pl.BlockSpec((1, _KNN_TR, K), lambda b, r: (b, r, 0)),
        out_shape=jax.ShapeDtypeStruct((B, N, K), jnp.int32),
    )(frows, frows, xx, xxt)


def _conv_body(z_ref, w_ref, y_ref, s_ref):
    i = pl.program_id(0)
    y = lax.dot_general(z_ref[...], w_ref[...], (((1,), (1,)), ((), ())),
                        preferred_element_type=jnp.float32)
    y_ref[...] = y
    ps = jnp.sum(y, axis=0, keepdims=True)
    pss = jnp.sum(y * y, axis=0, keepdims=True)
    row = lax.broadcasted_iota(jnp.int32, (8, y.shape[1]), 0)
    part = jnp.where(row == 0, ps, jnp.where(row == 1, pss, 0.0))

    @pl.when(i == 0)
    def _():
        s_ref[...] = jnp.zeros_like(s_ref)

    s_ref[...] += part


def _conv(z, w, tm=2048):
    cin, co = w.shape[1], w.shape[0]
    grid = (BN_ROWS // tm,)
    return pl.pallas_call(
        _conv_body,
        grid=grid,
        in_specs=[
            pl.BlockSpec((tm, cin), lambda i: (i, 0)),
            pl.BlockSpec((co, cin), lambda i: (0, 0)),
        ],
        out_specs=[
            pl.BlockSpec((tm, co), lambda i: (i, 0)),
            pl.BlockSpec((8, co), lambda i: (0, 0)),
        ],
        out_shape=[
            jax.ShapeDtypeStruct((BN_ROWS, co), jnp.float32),
            jax.ShapeDtypeStruct((8, co), jnp.float32),
        ],
    )(z, w)


def _conv5_body(z1_ref, z2_ref, z3_ref, z4_ref, w1_ref, w2_ref, w3_ref,
                w4_ref, y_ref, s_ref):
    i = pl.program_id(0)
    dn = (((1,), (1,)), ((), ()))
    y = lax.dot_general(z1_ref[...], w1_ref[...], dn,
                        preferred_element_type=jnp.float32)
    y += lax.dot_general(z2_ref[...], w2_ref[...], dn,
                         preferred_element_type=jnp.float32)
    y += lax.dot_general(z3_ref[...], w3_ref[...], dn,
                         preferred_element_type=jnp.float32)
    y += lax.dot_general(z4_ref[...], w4_ref[...], dn,
                         preferred_element_type=jnp.float32)
    y_ref[...] = y
    ps = jnp.sum(y, axis=0, keepdims=True)
    pss = jnp.sum(y * y, axis=0, keepdims=True)
    row = lax.broadcasted_iota(jnp.int32, (8, y.shape[1]), 0)
    part = jnp.where(row == 0, ps, jnp.where(row == 1, pss, 0.0))

    @pl.when(i == 0)
    def _():
        s_ref[...] = jnp.zeros_like(s_ref)

    s_ref[...] += part


def _conv5(z1, z2, z3, z4, w5, tm=512):
    co = w5.shape[0]
    w5a, w5b, w5c, w5d = w5[:, :128], w5[:, 128:256], w5[:, 256:384], w5[:, 384:]
    grid = (BN_ROWS // tm,)
    return pl.pallas_call(
        _conv5_body,
        grid=grid,
        in_specs=[
            pl.BlockSpec((tm, 128), lambda i: (i, 0)),
            pl.BlockSpec((tm, 128), lambda i: (i, 0)),
            pl.BlockSpec((tm, 128), lambda i: (i, 0)),
            pl.BlockSpec((tm, 256), lambda i: (i, 0)),
            pl.BlockSpec((co, 128), lambda i: (0, 0)),
            pl.BlockSpec((co, 128), lambda i: (0, 0)),
            pl.BlockSpec((co, 128), lambda i: (0, 0)),
            pl.BlockSpec((co, 256), lambda i: (0, 0)),
        ],
        out_specs=[
            pl.BlockSpec((tm, co), lambda i: (i, 0)),
            pl.BlockSpec((8, co), lambda i: (0, 0)),
        ],
        out_shape=[
            jax.ShapeDtypeStruct((BN_ROWS, co), jnp.float32),
            jax.ShapeDtypeStruct((8, co), jnp.float32),
        ],
    )(z1, z2, z3, z4, w5a, w5b, w5c, w5d)


def _var_body(y_ref, m_ref, s_ref):
    i = pl.program_id(0)
    dy = y_ref[...] - m_ref[...]
    pss = jnp.sum(dy * dy, axis=0, keepdims=True)
    row = lax.broadcasted_iota(jnp.int32, (8, dy.shape[1]), 0)
    part = jnp.where(row == 0, pss, 0.0)

    @pl.when(i == 0)
    def _():
        s_ref[...] = jnp.zeros_like(s_ref)

    s_ref[...] += part


def _varsum(y, mean, tm=2048):
    co = y.shape[1]
    grid = (BN_ROWS // tm,)
    return pl.pallas_call(
        _var_body,
        grid=grid,
        in_specs=[
            pl.BlockSpec((tm, co), lambda i: (i, 0)),
            pl.BlockSpec((1, co), lambda i: (0, 0)),
        ],
        out_specs=pl.BlockSpec((8, co), lambda i: (0, 0)),
        out_shape=jax.ShapeDtypeStruct((8, co), jnp.float32),
    )(y, mean.reshape(1, co))


def _bn_stats(y, s):
    mean = s[0] / BN_ROWS
    vs = _varsum(y, mean)
    var = vs[0] / BN_ROWS
    den = jnp.sqrt(var + EPS)
    return mean, den


_SC_R = 32


@functools.lru_cache(maxsize=None)
def _make_gather_max(co):
    yw = 128
    mesh = plsc.VectorSubcoreMesh(core_axis_name="c", subcore_axis_name="s")
    rows_per_w = BN_ROWS // 32
    n_chunks = rows_per_w // _SC_R
    idx_rows = _SC_R * K // 128

    idx_rows_w = rows_per_w * K // 128

    @functools.partial(
        pl.kernel,
        mesh=mesh,
        out_type=jax.ShapeDtypeStruct((BN_ROWS, 2 * co), jnp.float32),
        scratch_types=[
            pltpu.VMEM((idx_rows_w, 128), jnp.int32),
            pltpu.VMEM((_SC_R * K, yw), jnp.float32),
            pltpu.VMEM((_SC_R, yw), jnp.float32),
            pltpu.VMEM((_SC_R, 2 * co), jnp.float32),
            pltpu.VMEM((co,), jnp.float32),
            pltpu.VMEM((co,), jnp.float32),
            pltpu.VMEM((co,), jnp.float32),
            pltpu.VMEM((co,), jnp.float32),
            pltpu.SemaphoreType.DMA,
        ],
    )
    def gm(y_hbm, mn_hbm, dn_hbm, g_hbm, bb_hbm, idx_hbm, out_hbm,
           idx_v, gat_v, self_v, z_v, mn_v, dn_v, g_v, bb_v, sem):
        wid = lax.axis_index("s") * 2 + lax.axis_index("c")
        pltpu.sync_copy(mn_hbm, mn_v)
        pltpu.sync_copy(dn_hbm, dn_v)
        pltpu.sync_copy(g_hbm, g_v)
        pltpu.sync_copy(bb_hbm, bb_v)
        pltpu.sync_copy(idx_hbm.at[pl.ds(wid * idx_rows_w, idx_rows_w)], idx_v)

        def chunk(ci, carry):
            row0 = wid * rows_per_w + ci * _SC_R
            cps = []
            for i in range(idx_rows):
                cps.append(pltpu.async_copy(
                    y_hbm.at[idx_v.at[ci * idx_rows + i]],
                    gat_v.at[pl.ds(i * 128, 128)], sem))
            pltpu.sync_copy(y_hbm.at[pl.ds(row0, _SC_R)], self_v)
            for cp in cps:
                cp.wait()

            def row(r, c2):
                for l in range(co // 16):
                    sl = pl.ds(l * 16, 16)
                    acc = gat_v[r * K, sl]
                    for j in range(1, K):
                        acc = jnp.maximum(acc, gat_v[r * K + j, sl])
                    mn = mn_v[sl]
                    dn = dn_v[sl]
                    gg = g_v[sl]
                    bb = bb_v[sl]
                    fm = ((acc - mn) / dn) * gg + bb
                    fm = jnp.where(fm >= 0.0, fm, 0.2 * fm)
                    fs = ((self_v[r, sl] - mn) / dn) * gg + bb
                    fs = jnp.where(fs >= 0.0, fs, 0.2 * fs)
                    z_v[r, sl] = fs
                    z_v[r, pl.ds(co + l * 16, 16)] = fm - fs
                return c2

            lax.fori_loop(0, _SC_R, row, 0)
            pltpu.sync_copy(z_v, out_hbm.at[pl.ds(row0, _SC_R)])
            return carry

        lax.fori_loop(0, n_chunks, chunk, 0)

    return gm


def _colmax_body(y_ref, mn_ref, dn_ref, g_ref, b_ref, o_ref):
    a = ((y_ref[...] - mn_ref[...]) / dn_ref[...]) * g_ref[...] + b_ref[...]
    m = jnp.max(a, axis=1, keepdims=True)
    m = jnp.where(m >= 0.0, m, 0.2 * m)
    o_ref[...] = jnp.broadcast_to(m, (m.shape[0], 8))


def _colmax(y5, mn, dn, g, bb, tm=1024):
    grid = (BN_ROWS // tm,)
    vec = pl.BlockSpec((1, 1024), lambda i: (0, 0))
    return pl.pallas_call(
        _colmax_body,
        grid=grid,
        in_specs=[pl.BlockSpec((tm, 1024), lambda i: (i, 0)), vec, vec, vec, vec],
        out_specs=pl.BlockSpec((tm, 8), lambda i: (i, 0)),
        out_shape=jax.ShapeDtypeStruct((BN_ROWS, 8), jnp.float32),
    )(y5, mn.reshape(1, -1), dn.reshape(1, -1), g.reshape(1, -1), bb.reshape(1, -1))


def _fc_body(h_ref, l1_ref, g6_ref, b6_ref, l2_ref, l2b_ref, g7_ref, b7_ref,
             l3_ref, l3b_ref, o_ref):
    dn = (((1,), (1,)), ((), ()))

    def bn_lrelu(y, g, bb):
        mean = jnp.mean(y, axis=0, keepdims=True)
        var = jnp.mean((y - mean) * (y - mean), axis=0, keepdims=True)
        a = (y - mean) / jnp.sqrt(var + EPS) * g + bb
        return jnp.where(a >= 0.0, a, 0.2 * a)

    h1 = bn_lrelu(lax.dot_general(h_ref[...], l1_ref[...], dn,
                                  preferred_element_type=jnp.float32),
                  g6_ref[...], b6_ref[...])
    h2 = bn_lrelu(lax.dot_general(h1, l2_ref[...], dn,
                                  preferred_element_type=jnp.float32)
                  + l2b_ref[...],
                  g7_ref[...], b7_ref[...])
    o_ref[...] = lax.dot_general(h2, l3_ref[...], dn,
                                 preferred_element_type=jnp.float32) + l3b_ref[...]


def _fc(h, l1, g6, b6, l2, l2b, g7, b7, l3, l3b):
    return pl.pallas_call(
        _fc_body,
        out_shape=jax.ShapeDtypeStruct((B, 40), jnp.float32),
    )(h, l1, g6.reshape(1, -1), b6.reshape(1, -1), l2, l2b.reshape(1, -1),
      g7.reshape(1, -1), b7.reshape(1, -1), l3, l3b.reshape(1, -1))


def kernel(x, W1, g1, b1, W2, g2, b2, W3, g3, b3, W4, g4, b4, W5, g5, b5,
           L1, g6, b6, L2, L2b, g7, b7, L3, L3b):
    xr = jnp.transpose(x, (0, 2, 1))
    xp = jnp.pad(xr, ((0, 0), (0, 0), (0, CP - 3)))
    rows0 = xp.reshape(BN_ROWS, CP)
    w1p = jnp.pad(W1, ((0, 0), (0, CP - 3)))

    def layer(frows, zrows, w, g, bta, gm, xx):
        co = w.shape[0]
        idx = _knn(frows, xx).reshape(-1, 128)
        wp = w if co == 128 else jnp.pad(w, ((0, 128 - co), (0, 0)))
        y, s = _conv(zrows, wp)
        mean, den = _bn_stats(y, s)
        return gm(y, mean[:co], den[:co], g, bta, idx)

    def xx_of(z):
        return jnp.sum(z * z, axis=1).reshape(B, 1, N)

    gm64, gm128 = _make_gather_max(64), _make_gather_max(128)
    xx1 = jnp.sum(x ** 2, axis=1, keepdims=True)
    z1 = layer(xp, rows0, w1p, g1, b1, gm64, xx1)
    z2 = layer(z1.reshape(B, N, CP), z1, W2, g2, b2, gm64, xx_of(z1))
    z3 = layer(z2.reshape(B, N, CP), z2, W3, g3, b3, gm64, xx_of(z2))
    z4 = layer(z3.reshape(B, N, CP), z3, W4, g4, b4, gm128, xx_of(z3))

    y5, s5 = _conv5(z1, z2, z3, z4, W5)
    mean5, den5 = _bn_stats(y5, s5)
    hm = _colmax(y5, mean5, den5, g5, b5)
    h = hm[:, 0].reshape(B, N)

    return _fc(h, L1, g6, b6, L2, L2b, g7, b7, L3, L3b)
